# baseline (speedup 1.0000x reference)
import dataclasses
import functools

import jax
import jax.numpy as jnp
import numpy as np
from jax import lax
from jax.experimental import pallas as pl
from jax.experimental.pallas import tpu as pltpu
from jax.experimental.pallas import tpu_sc as plsc

_B = 64
_N_IN = 16384
_N_OUT = 16384
_NC = 2
_NS = 16
_NW = _NC * _NS
_W = 128
_LANES = 16

_DEPTH = 3
_PHASES = 2


def _feature_shuffle():
    sigma = np.empty((_B,), np.int32)
    for h in range(2):
        for k in range(16):
            sigma[h * 32 + 2 * k] = h * 32 + k
            sigma[h * 32 + 2 * k + 1] = h * 32 + 16 + k
    return sigma


_SIGMA = _feature_shuffle()


def _sc_body(ch, rows_hbm, cols_hbm, vals_hbm, xt_hbm, part_hbm,
             acc, rows_v, cols_v, vals_v,
             g0, g1, g2, s0, s1, s2,
             gsem0, gsem1, gsem2, ssem0, ssem1, ssem2):
    gbufs = (g0, g1, g2)
    sbufs = (s0, s1, s2)
    gsems = (gsem0, gsem1, gsem2)
    ssems = (ssem0, ssem1, ssem2)
    cid = lax.axis_index("c")
    sid = lax.axis_index("s")
    wid = sid * _NC + cid
    ph_ch = ch // _PHASES
    nblk = ph_ch // _DEPTH

    zero16 = jnp.zeros((_LANES,), jnp.float32)

    @pl.loop(0, _W)
    def _(i):
        for f in range(_B // _LANES):
            s0[i, pl.ds(f * _LANES, _LANES)] = zero16

    rows_per_tile = _N_OUT // _NS

    @pl.loop(0, rows_per_tile // _W)
    def _(t):
        pltpu.sync_copy(s0, acc.at[pl.ds(sid * rows_per_tile + t * _W, _W)])

    plsc.subcore_barrier()

    dnums = lax.GatherDimensionNumbers(
        offset_dims=(), collapsed_slice_dims=(0,), start_index_map=(0,))
    himask = jnp.int32(-65536)

    @pl.loop(0, _PHASES)
    def _(ph):
        base = ph * ph_ch
        pltpu.sync_copy(rows_hbm.at[wid, pl.ds(base, ph_ch)], rows_v)
        pltpu.sync_copy(cols_hbm.at[wid, pl.ds(base, ph_ch)], cols_v)
        pltpu.sync_copy(vals_hbm.at[wid, pl.ds(base, ph_ch)], vals_v)

        for b in range(_DEPTH):
            pltpu.async_copy(xt_hbm.at[rows_v.at[b]], gbufs[b], gsems[b])

        @pl.loop(0, nblk)
        def _(blk):
            for b in range(_DEPTH):
                c = blk * _DEPTH + b

                @pl.when(blk > 0)
                def _():
                    pltpu.make_async_copy(
                        part_hbm.at[cid, pl.ds(0, _W)], sbufs[b],
                        ssems[b]).wait()

                pltpu.make_async_copy(
                    xt_hbm.at[pl.ds(0, _W)], gbufs[b], gsems[b]).wait()

                @plsc.parallel_loop(0, _W // _LANES, unroll=2)
                def _(g):
                    v16 = vals_v[c, pl.ds(g * _LANES, _LANES)]
                    for j in range(_LANES):
                        bv = lax.gather(
                            v16, jnp.full((_LANES, 1), j, jnp.int32), dnums,
                            (1,), mode=lax.GatherScatterMode.PROMISE_IN_BOUNDS)
                        row = g * _LANES + j
                        for h in range(2):
                            w = gbufs[b][row, pl.ds(h * _LANES, _LANES)]
                            lo = plsc.bitcast(w << 16, jnp.float32)
                            hi = plsc.bitcast(w & himask, jnp.float32)
                            sbufs[b][row, pl.ds(h * 32, _LANES)] = lo * bv
                            sbufs[b][row, pl.ds(h * 32 + 16, _LANES)] = hi * bv

                @pl.when(blk < nblk - 1)
                def _():
                    pltpu.async_copy(
                        xt_hbm.at[rows_v.at[c + _DEPTH]], gbufs[b], gsems[b])

                pltpu.async_copy(sbufs[b], acc.at[cols_v.at[c]], ssems[b],
                                 add=True)

        for b in range(_DEPTH):
            pltpu.make_async_copy(part_hbm.at[cid, pl.ds(0, _W)], sbufs[b],
                                  ssems[b]).wait()

    plsc.subcore_barrier()

    pltpu.sync_copy(acc.at[pl.ds(sid * rows_per_tile, rows_per_tile)],
                    part_hbm.at[cid, pl.ds(sid * rows_per_tile, rows_per_tile)])


def _compiler_params():
    cp = pltpu.CompilerParams(use_tc_tiling_on_sc=False)
    if "needs_layout_passes" in pltpu.CompilerParams.__dataclass_fields__:
        cp = dataclasses.replace(cp, needs_layout_passes=False)
    return cp


def _sc_spmm(rows3, cols3, vals3, xt_i32):
    ch = rows3.shape[1]
    mesh = plsc.VectorSubcoreMesh(core_axis_name="c", subcore_axis_name="s")
    kern = pl.kernel(
        functools.partial(_sc_body, ch),
        out_type=jax.ShapeDtypeStruct((_NC, _N_OUT, _B), jnp.float32),
        mesh=mesh,
        scratch_types=(
            [
                pltpu.VMEM_SHARED((_N_OUT, _B), jnp.float32),
                pltpu.VMEM((ch // _PHASES, _W), jnp.int32),
                pltpu.VMEM((ch // _PHASES, _W), jnp.int32),
                pltpu.VMEM((ch // _PHASES, _W), jnp.float32),
            ]
            + [pltpu.VMEM((_W, _B // 2), jnp.int32) for _ in range(_DEPTH)]
            + [pltpu.VMEM((_W, _B), jnp.float32) for _ in range(_DEPTH)]
            + [pltpu.SemaphoreType.DMA for _ in range(2 * _DEPTH)]
        ),
        compiler_params=_compiler_params(),
    )
    return kern(rows3, cols3, vals3, xt_i32)


def _combine(parts):
    blk = 2048

    def body(p_ref, o_ref):
        s = p_ref[0] + p_ref[1]
        o_ref[...] = s.T

    return pl.pallas_call(
        body,
        grid=(_N_OUT // blk,),
        in_specs=[pl.BlockSpec((2, blk, _B), lambda i: (0, i, 0))],
        out_specs=pl.BlockSpec((_B, blk), lambda i: (0, i)),
        out_shape=jax.ShapeDtypeStruct((_B, _N_OUT), jnp.float32),
    )(parts)


def kernel(x, vals, rows, cols):
    nnz = vals.shape[0]
    ch = -(-nnz // (_NW * _W))
    q = _DEPTH * _PHASES
    ch = -(-ch // q) * q
    padded = _NW * _W * ch
    pad = padded - nnz
    ar = jnp.arange(pad, dtype=jnp.int32)
    rows_p = jnp.concatenate([rows.astype(jnp.int32), ar % _N_IN])
    cols_p = jnp.concatenate([cols.astype(jnp.int32), ar % _N_OUT])
    vals_p = jnp.concatenate([vals, jnp.zeros((pad,), vals.dtype)])
    rows3 = rows_p.reshape(_NW, ch, _W)
    cols3 = cols_p.reshape(_NW, ch, _W)
    vals3 = vals_p.reshape(_NW, ch, _W)
    xtb = x.T[:, _SIGMA].astype(jnp.bfloat16)
    xt_i32 = lax.bitcast_convert_type(
        xtb.reshape(_N_IN, _B // 2, 2), jnp.int32)
    parts = _sc_spmm(rows3, cols3, vals3, xt_i32)
    return _combine(parts)


# speedup vs baseline: 1.3978x; 1.3978x over previous
import functools

import jax
import jax.numpy as jnp
from jax import lax
from jax.experimental import pallas as pl
from jax.experimental.pallas import tpu as pltpu
from jax.experimental.pallas import tpu_sc as plsc

_B = 64
_N_IN = 16384
_N_OUT = 16384
_NC = 2
_NS = 16
_NW = _NC * _NS
_W = 128
_LANES = 16

_DEPTH = 3
_PHASES = 2


def _sc_body(ch, rows_hbm, cols_hbm, vals_hbm, xt_hbm, part_hbm,
             acc, rows_v, cols_v, vals_v,
             g0, g1, g2, s0, s1, s2,
             gsem0, gsem1, gsem2, ssem0, ssem1, ssem2):
    gbufs = (g0, g1, g2)
    sbufs = (s0, s1, s2)
    gsems = (gsem0, gsem1, gsem2)
    ssems = (ssem0, ssem1, ssem2)
    cid = lax.axis_index("c")
    sid = lax.axis_index("s")
    wid = sid * _NC + cid
    ph_ch = ch // _PHASES
    nblk = ph_ch // _DEPTH

    zero16 = jnp.zeros((_LANES,), jnp.float32)

    @pl.loop(0, _W)
    def _(i):
        for f in range(_B // _LANES):
            s0[i, pl.ds(f * _LANES, _LANES)] = zero16

    rows_per_tile = _N_OUT // _NS

    @pl.loop(0, rows_per_tile // _W)
    def _(t):
        pltpu.sync_copy(s0, acc.at[pl.ds(sid * rows_per_tile + t * _W, _W)])

    plsc.subcore_barrier()

    dnums = lax.GatherDimensionNumbers(
        offset_dims=(), collapsed_slice_dims=(0,), start_index_map=(0,))

    @pl.loop(0, _PHASES)
    def _(ph):
        base = ph * ph_ch
        pltpu.sync_copy(rows_hbm.at[wid, pl.ds(base, ph_ch)], rows_v)
        pltpu.sync_copy(cols_hbm.at[wid, pl.ds(base, ph_ch)], cols_v)
        pltpu.sync_copy(vals_hbm.at[wid, pl.ds(base, ph_ch)], vals_v)

        for b in range(_DEPTH):
            pltpu.async_copy(xt_hbm.at[rows_v.at[b]], gbufs[b], gsems[b])

        @pl.loop(0, nblk)
        def _(blk):
            for b in range(_DEPTH):
                c = blk * _DEPTH + b

                @pl.when(blk > 0)
                def _():
                    pltpu.make_async_copy(
                        xt_hbm.at[pl.ds(0, _W)], sbufs[b], ssems[b]).wait()

                pltpu.make_async_copy(
                    xt_hbm.at[pl.ds(0, _W)], gbufs[b], gsems[b]).wait()

                @plsc.parallel_loop(0, _W // _LANES, unroll=2)
                def _(g):
                    v16 = vals_v[c, pl.ds(g * _LANES, _LANES)]
                    for j in range(_LANES):
                        bv = lax.gather(
                            v16, jnp.full((_LANES, 1), j, jnp.int32), dnums,
                            (1,), mode=lax.GatherScatterMode.PROMISE_IN_BOUNDS)
                        for f in range(_B // _LANES):
                            idx = (g * _LANES + j, pl.ds(f * _LANES, _LANES))
                            sbufs[b][idx] = gbufs[b][idx] * bv

                @pl.when(blk < nblk - 1)
                def _():
                    pltpu.async_copy(
                        xt_hbm.at[rows_v.at[c + _DEPTH]], gbufs[b], gsems[b])

                pltpu.async_copy(sbufs[b], acc.at[cols_v.at[c]], ssems[b],
                                 add=True)

        for b in range(_DEPTH):
            pltpu.make_async_copy(xt_hbm.at[pl.ds(0, _W)], sbufs[b],
                                  ssems[b]).wait()

    plsc.subcore_barrier()

    pltpu.sync_copy(acc.at[pl.ds(sid * rows_per_tile, rows_per_tile)],
                    part_hbm.at[cid, pl.ds(sid * rows_per_tile, rows_per_tile)])


def _sc_spmm(rows3, cols3, vals3, xt):
    ch = rows3.shape[1]
    mesh = plsc.VectorSubcoreMesh(core_axis_name="c", subcore_axis_name="s")
    kern = pl.kernel(
        functools.partial(_sc_body, ch),
        out_type=jax.ShapeDtypeStruct((_NC, _N_OUT, _B), jnp.float32),
        mesh=mesh,
        scratch_types=(
            [
                pltpu.VMEM_SHARED((_N_OUT, _B), jnp.float32),
                pltpu.VMEM((ch // _PHASES, _W), jnp.int32),
                pltpu.VMEM((ch // _PHASES, _W), jnp.int32),
                pltpu.VMEM((ch // _PHASES, _W), jnp.float32),
            ]
            + [pltpu.VMEM((_W, _B), jnp.float32) for _ in range(2 * _DEPTH)]
            + [pltpu.SemaphoreType.DMA for _ in range(2 * _DEPTH)]
        ),
        compiler_params=pltpu.CompilerParams(use_tc_tiling_on_sc=False),
    )
    return kern(rows3, cols3, vals3, xt)


def _combine(parts):
    rows = _N_OUT * _B // 128
    pv = parts.reshape(_NC, rows, 128)
    blk = 2048

    def body(p_ref, o_ref):
        o_ref[...] = p_ref[0] + p_ref[1]

    summed = pl.pallas_call(
        body,
        grid=(rows // blk,),
        in_specs=[pl.BlockSpec((2, blk, 128), lambda i: (0, i, 0))],
        out_specs=pl.BlockSpec((blk, 128), lambda i: (i, 0)),
        out_shape=jax.ShapeDtypeStruct((rows, 128), jnp.float32),
    )(pv)
    r3 = summed.reshape(rows, 2, _B)
    return r3.transpose(2, 0, 1).reshape(_B, _N_OUT)


def kernel(x, vals, rows, cols):
    nnz = vals.shape[0]
    ch = -(-nnz // (_NW * _W))
    q = _DEPTH * _PHASES
    ch = -(-ch // q) * q
    padded = _NW * _W * ch
    pad = padded - nnz
    ar = jnp.arange(pad, dtype=jnp.int32)
    rows_p = jnp.concatenate([rows.astype(jnp.int32), ar % _N_IN])
    cols_p = jnp.concatenate([cols.astype(jnp.int32), ar % _N_OUT])
    vals_p = jnp.concatenate([vals, jnp.zeros((pad,), vals.dtype)])
    rows3 = rows_p.reshape(_NW, ch, _W)
    cols3 = cols_p.reshape(_NW, ch, _W)
    vals3 = vals_p.reshape(_NW, ch, _W)
    xt = x.T
    parts = _sc_spmm(rows3, cols3, vals3, xt)
    return _combine(parts)


# speedup vs baseline: 1.5251x; 1.0911x over previous
import functools

import jax
import jax.numpy as jnp
from jax import lax
from jax.experimental import pallas as pl
from jax.experimental.pallas import tpu as pltpu
from jax.experimental.pallas import tpu_sc as plsc

_B = 64
_N_IN = 16384
_N_OUT = 16384
_NC = 2
_NS = 16
_NW = _NC * _NS
_W = 128
_LANES = 16

_DEPTH = 3
_PHASES = 2


def _sc_body(ch, rows_hbm, cols_hbm, vals_hbm, xt_hbm, part_hbm,
             acc, rows_v, cols_v, vals_v,
             g0, g1, g2, s0, s1, s2,
             gsem0, gsem1, gsem2, ssem0, ssem1, ssem2):
    gbufs = (g0, g1, g2)
    sbufs = (s0, s1, s2)
    gsems = (gsem0, gsem1, gsem2)
    ssems = (ssem0, ssem1, ssem2)
    cid = lax.axis_index("c")
    sid = lax.axis_index("s")
    wid = sid * _NC + cid
    ph_ch = ch // _PHASES
    nblk = ph_ch // _DEPTH

    zero16 = jnp.zeros((_LANES,), jnp.float32)

    @pl.loop(0, _W)
    def _(i):
        for f in range(_B // _LANES):
            s0[i, pl.ds(f * _LANES, _LANES)] = zero16

    rows_per_tile = _N_OUT // _NS

    @pl.loop(0, rows_per_tile // _W)
    def _(t):
        pltpu.sync_copy(s0, acc.at[pl.ds(sid * rows_per_tile + t * _W, _W)])

    plsc.subcore_barrier()

    dnums = lax.GatherDimensionNumbers(
        offset_dims=(), collapsed_slice_dims=(0,), start_index_map=(0,))

    @pl.loop(0, _PHASES)
    def _(ph):
        base = ph * ph_ch
        pltpu.sync_copy(rows_hbm.at[wid, pl.ds(base, ph_ch)], rows_v)
        pltpu.sync_copy(cols_hbm.at[wid, pl.ds(base, ph_ch)], cols_v)
        pltpu.sync_copy(vals_hbm.at[wid, pl.ds(base, ph_ch)], vals_v)

        for b in range(_DEPTH):
            pltpu.async_copy(xt_hbm.at[rows_v.at[b]], gbufs[b], gsems[b])

        @pl.loop(0, nblk)
        def _(blk):
            for b in range(_DEPTH):
                c = blk * _DEPTH + b

                @pl.when(blk > 0)
                def _():
                    pltpu.make_async_copy(
                        xt_hbm.at[pl.ds(0, _W)], sbufs[b], ssems[b]).wait()

                pltpu.make_async_copy(
                    xt_hbm.at[pl.ds(0, _W)], gbufs[b], gsems[b]).wait()

                @plsc.parallel_loop(0, _W // _LANES, unroll=2)
                def _(g):
                    v16 = vals_v[c, pl.ds(g * _LANES, _LANES)]
                    for j in range(_LANES):
                        bv = lax.gather(
                            v16, jnp.full((_LANES, 1), j, jnp.int32), dnums,
                            (1,), mode=lax.GatherScatterMode.PROMISE_IN_BOUNDS)
                        for f in range(_B // _LANES):
                            idx = (g * _LANES + j, pl.ds(f * _LANES, _LANES))
                            sbufs[b][idx] = gbufs[b][idx] * bv

                @pl.when(blk < nblk - 1)
                def _():
                    pltpu.async_copy(
                        xt_hbm.at[rows_v.at[c + _DEPTH]], gbufs[b], gsems[b])

                pltpu.async_copy(sbufs[b], acc.at[cols_v.at[c]], ssems[b],
                                 add=True)

        for b in range(_DEPTH):
            pltpu.make_async_copy(xt_hbm.at[pl.ds(0, _W)], sbufs[b],
                                  ssems[b]).wait()

    plsc.subcore_barrier()

    pltpu.sync_copy(acc.at[pl.ds(sid * rows_per_tile, rows_per_tile)],
                    part_hbm.at[cid, pl.ds(sid * rows_per_tile, rows_per_tile)])


def _sc_spmm(rows3, cols3, vals3, xt):
    ch = rows3.shape[1]
    mesh = plsc.VectorSubcoreMesh(core_axis_name="c", subcore_axis_name="s")
    kern = pl.kernel(
        functools.partial(_sc_body, ch),
        out_type=jax.ShapeDtypeStruct((_NC, _N_OUT, _B), jnp.float32),
        mesh=mesh,
        scratch_types=(
            [
                pltpu.VMEM_SHARED((_N_OUT, _B), jnp.float32),
                pltpu.VMEM((ch // _PHASES, _W), jnp.int32),
                pltpu.VMEM((ch // _PHASES, _W), jnp.int32),
                pltpu.VMEM((ch // _PHASES, _W), jnp.float32),
            ]
            + [pltpu.VMEM((_W, _B), jnp.float32) for _ in range(2 * _DEPTH)]
            + [pltpu.SemaphoreType.DMA for _ in range(2 * _DEPTH)]
        ),
        compiler_params=pltpu.CompilerParams(use_tc_tiling_on_sc=False),
    )
    return kern(rows3, cols3, vals3, xt)


def _combine(parts):
    rows = _N_OUT * _B // 128
    pv = parts.reshape(_NC, rows, 128)
    blk = 1024

    def body(p_ref, o_ref):
        s = p_ref[0] + p_ref[1]
        o_ref[:, :blk] = s[:, :_B].T
        o_ref[:, blk:] = s[:, _B:].T

    return pl.pallas_call(
        body,
        grid=(rows // blk,),
        in_specs=[pl.BlockSpec((2, blk, 128), lambda i: (0, i, 0))],
        out_specs=pl.BlockSpec((_B, 2 * blk), lambda i: (0, i)),
        out_shape=jax.ShapeDtypeStruct((_B, _N_OUT), jnp.float32),
    )(pv)


def kernel(x, vals, rows, cols):
    nnz = vals.shape[0]
    ch = -(-nnz // (_NW * _W))
    q = _DEPTH * _PHASES
    ch = -(-ch // q) * q
    padded = _NW * _W * ch
    pad = padded - nnz
    ar = jnp.arange(pad, dtype=jnp.int32)
    rows_p = jnp.concatenate([rows.astype(jnp.int32), ar % _N_IN])
    cols_p = jnp.concatenate([cols.astype(jnp.int32), ar % _N_OUT])
    cols_p = ((cols_p // 2048) * 2048 + 2 * (cols_p % 1024)
              + (cols_p % 2048) // 1024)
    vals_p = jnp.concatenate([vals, jnp.zeros((pad,), vals.dtype)])
    rows3 = rows_p.reshape(_NW, ch, _W)
    cols3 = cols_p.reshape(_NW, ch, _W)
    vals3 = vals_p.reshape(_NW, ch, _W)
    xt = x.T
    parts = _sc_spmm(rows3, cols3, vals3, xt)
    return _combine(parts)


# speedup vs baseline: 1.5281x; 1.0020x over previous
import functools

import jax
import jax.numpy as jnp
from jax import lax
from jax.experimental import pallas as pl
from jax.experimental.pallas import tpu as pltpu
from jax.experimental.pallas import tpu_sc as plsc

_B = 64
_N_IN = 16384
_N_OUT = 16384
_NC = 2
_NS = 16
_NW = _NC * _NS
_W = 128
_LANES = 16

_DEPTH = 3
_PHASES = 2


def _sc_body(ch, rows_hbm, cols_hbm, vals_hbm, xt_hbm, part_hbm,
             acc, rows_v, cols_v, vals_v,
             g0, g1, g2, s0, s1, s2,
             gsem0, gsem1, gsem2, ssem0, ssem1, ssem2):
    gbufs = (g0, g1, g2)
    sbufs = (s0, s1, s2)
    gsems = (gsem0, gsem1, gsem2)
    ssems = (ssem0, ssem1, ssem2)
    cid = lax.axis_index("c")
    sid = lax.axis_index("s")
    wid = sid * _NC + cid
    ph_ch = ch // _PHASES
    nblk = ph_ch // _DEPTH
    shard0 = wid * ch * _W

    zero16 = jnp.zeros((_LANES,), jnp.float32)

    @pl.loop(0, _W)
    def _(i):
        for f in range(_B // _LANES):
            s0[i, pl.ds(f * _LANES, _LANES)] = zero16

    rows_per_tile = _N_OUT // _NS

    @pl.loop(0, rows_per_tile // _W)
    def _(t):
        pltpu.sync_copy(s0, acc.at[pl.ds(sid * rows_per_tile + t * _W, _W)])

    plsc.subcore_barrier()

    dnums = lax.GatherDimensionNumbers(
        offset_dims=(), collapsed_slice_dims=(0,), start_index_map=(0,))

    @pl.loop(0, _PHASES)
    def _(ph):
        base = ph * ph_ch
        pltpu.sync_copy(rows_hbm.at[pl.ds(shard0 + base * _W, ph_ch * _W)],
                        rows_v)
        pltpu.sync_copy(cols_hbm.at[wid, pl.ds(base, ph_ch)], cols_v)
        pltpu.sync_copy(vals_hbm.at[pl.ds(shard0 + base * _W, ph_ch * _W)],
                        vals_v)

        for b in range(_DEPTH):
            pltpu.async_copy(xt_hbm.at[rows_v.at[pl.ds(b * _W, _W)]],
                             gbufs[b], gsems[b])

        @pl.loop(0, nblk)
        def _(blk):
            for b in range(_DEPTH):
                c = blk * _DEPTH + b

                @pl.when(blk > 0)
                def _():
                    pltpu.make_async_copy(
                        xt_hbm.at[pl.ds(0, _W)], sbufs[b], ssems[b]).wait()

                pltpu.make_async_copy(
                    xt_hbm.at[pl.ds(0, _W)], gbufs[b], gsems[b]).wait()

                @plsc.parallel_loop(0, _W // _LANES, unroll=2)
                def _(g):
                    v16 = vals_v[pl.ds(c * _W + g * _LANES, _LANES)]
                    for j in range(_LANES):
                        bv = lax.gather(
                            v16, jnp.full((_LANES, 1), j, jnp.int32), dnums,
                            (1,), mode=lax.GatherScatterMode.PROMISE_IN_BOUNDS)
                        for f in range(_B // _LANES):
                            idx = (g * _LANES + j, pl.ds(f * _LANES, _LANES))
                            sbufs[b][idx] = gbufs[b][idx] * bv

                @pl.when(blk < nblk - 1)
                def _():
                    pltpu.async_copy(
                        xt_hbm.at[rows_v.at[pl.ds((c + _DEPTH) * _W, _W)]],
                        gbufs[b], gsems[b])

                pltpu.async_copy(sbufs[b], acc.at[cols_v.at[c]], ssems[b],
                                 add=True)

        for b in range(_DEPTH):
            pltpu.make_async_copy(xt_hbm.at[pl.ds(0, _W)], sbufs[b],
                                  ssems[b]).wait()

    plsc.subcore_barrier()

    pltpu.sync_copy(acc.at[pl.ds(sid * rows_per_tile, rows_per_tile)],
                    part_hbm.at[cid, pl.ds(sid * rows_per_tile, rows_per_tile)])


def _sc_spmm(rows_flat, cols3, vals_flat, xt):
    ch = cols3.shape[1]
    mesh = plsc.VectorSubcoreMesh(core_axis_name="c", subcore_axis_name="s")
    kern = pl.kernel(
        functools.partial(_sc_body, ch),
        out_type=jax.ShapeDtypeStruct((_NC, _N_OUT, _B), jnp.float32),
        mesh=mesh,
        scratch_types=(
            [
                pltpu.VMEM_SHARED((_N_OUT, _B), jnp.float32),
                pltpu.VMEM((ch // _PHASES * _W,), jnp.int32),
                pltpu.VMEM((ch // _PHASES, _W), jnp.int32),
                pltpu.VMEM((ch // _PHASES * _W,), jnp.float32),
            ]
            + [pltpu.VMEM((_W, _B), jnp.float32) for _ in range(2 * _DEPTH)]
            + [pltpu.SemaphoreType.DMA for _ in range(2 * _DEPTH)]
        ),
        compiler_params=pltpu.CompilerParams(use_tc_tiling_on_sc=False),
    )
    return kern(rows_flat, cols3, vals_flat, xt)


def _combine(parts):
    rows = _N_OUT * _B // 128
    pv = parts.reshape(_NC, rows, 128)
    blk = 1024

    def body(p_ref, o_ref):
        s = p_ref[0] + p_ref[1]
        o_ref[:, :blk] = s[:, :_B].T
        o_ref[:, blk:] = s[:, _B:].T

    return pl.pallas_call(
        body,
        grid=(rows // blk,),
        in_specs=[pl.BlockSpec((2, blk, 128), lambda i: (0, i, 0))],
        out_specs=pl.BlockSpec((_B, 2 * blk), lambda i: (0, i)),
        out_shape=jax.ShapeDtypeStruct((_B, _N_OUT), jnp.float32),
    )(pv)


def kernel(x, vals, rows, cols):
    nnz = vals.shape[0]
    ch = -(-nnz // (_NW * _W))
    q = _DEPTH * _PHASES
    ch = -(-ch // q) * q
    padded = _NW * _W * ch
    pad = padded - nnz
    ar = jnp.arange(pad, dtype=jnp.int32)
    rows_p = jnp.concatenate([rows.astype(jnp.int32), ar % _N_IN])
    cols_p = jnp.concatenate([cols.astype(jnp.int32), ar % _N_OUT])
    cols_p = ((cols_p // 2048) * 2048 + 2 * (cols_p % 1024)
              + (cols_p % 2048) // 1024)
    vals_p = jnp.concatenate([vals, jnp.zeros((pad,), vals.dtype)])
    cols3 = cols_p.reshape(_NW, ch, _W)
    xt = x.T
    parts = _sc_spmm(rows_p, cols3, vals_p, xt)
    return _combine(parts)
